# baseline (speedup 1.0000x reference)
import jax
import jax.numpy as jnp
from jax.experimental import pallas as pl
from jax.experimental.pallas import tpu as pltpu

_GS = 16
_K = _GS + 1
_S = 128


def _knn_body(xq_ref, xr_ref, dist_ref, idx_ref):
    b = pl.program_id(0)
    xq = xq_ref[0]
    xr = xr_ref[0]
    Q = xq.shape[0]
    N = xr.shape[1]
    C = N // _S

    q0 = xq[:, 0:1]
    q1 = xq[:, 1:2]
    q2 = xq[:, 2:3]
    sq_q = q0 * q0 + q1 * q1 + q2 * q2
    inner = jnp.dot(xq.astype(jnp.bfloat16), xr.astype(jnp.bfloat16),
                    preferred_element_type=jnp.float32)

    inf = jnp.float32(jnp.inf)
    big = jnp.int32(N)
    lane = jax.lax.broadcasted_iota(jnp.int32, (1, _S), 1)
    cidx = jax.lax.broadcasted_iota(jnp.int32, (1, C), 1)

    chunks = []
    cms = []
    acms = []
    for c in range(C):
        r = xr[:, c * _S:(c + 1) * _S]
        r0 = r[0:1, :]
        r1 = r[1:2, :]
        r2 = r[2:3, :]
        sq_r = r0 * r0 + r1 * r1 + r2 * r2
        innr = inner[:, c * _S:(c + 1) * _S]
        d2c = jnp.maximum(sq_q + sq_r - 2.0 * innr, 0.0)
        gcol = lane + c * _S
        m = jnp.min(d2c, axis=1, keepdims=True)
        a = jnp.min(jnp.where(d2c == m, gcol, big), axis=1, keepdims=True)
        chunks.append(d2c)
        cms.append(m)
        acms.append(a)
    cm = jnp.concatenate(cms, axis=1)
    acm = jnp.concatenate(acms, axis=1)

    dists = []
    idxs = []
    for j in range(_K):
        m = jnp.min(cm, axis=1, keepdims=True)
        cstar = jnp.min(jnp.where(cm == m, cidx, jnp.int32(C)),
                        axis=1, keepdims=True)
        amin = jnp.min(jnp.where(cidx == cstar, acm, big),
                       axis=1, keepdims=True)
        if j > 0:
            dists.append(jnp.sqrt(m))
            idxs.append(amin)
        if j == _K - 1:
            break
        acc = chunks[0]
        for c in range(1, C):
            acc = jnp.where(cstar == c, chunks[c], acc)
        gcol = cstar * _S + lane
        keep = (acc > m) | ((acc == m) & (gcol > amin))
        eff = jnp.where(keep, acc, inf)
        nm = jnp.min(eff, axis=1, keepdims=True)
        na = jnp.min(jnp.where(eff == nm, gcol, big), axis=1, keepdims=True)
        sel = cidx == cstar
        cm = jnp.where(sel, nm, cm)
        acm = jnp.where(sel, na, acm)
    dist_ref[0] = jnp.concatenate(dists, axis=1)
    idx_ref[0] = jnp.concatenate(idxs, axis=1) + b * N


def kernel(xyz):
    B, N, _ = xyz.shape
    Q = 512
    xt = xyz.transpose(0, 2, 1)
    dist, idx = pl.pallas_call(
        _knn_body,
        grid=(B, N // Q),
        in_specs=[
            pl.BlockSpec((1, Q, 3), lambda b, q: (b, q, 0)),
            pl.BlockSpec((1, 3, N), lambda b, q: (b, 0, 0)),
        ],
        out_specs=[
            pl.BlockSpec((1, Q, _GS), lambda b, q: (b, q, 0)),
            pl.BlockSpec((1, Q, _GS), lambda b, q: (b, q, 0)),
        ],
        out_shape=[
            jax.ShapeDtypeStruct((B, N, _GS), jnp.float32),
            jax.ShapeDtypeStruct((B, N, _GS), jnp.int32),
        ],
        compiler_params=pltpu.CompilerParams(
            dimension_semantics=("parallel", "parallel")),
    )(xyz, xt)
    return dist, idx.reshape(-1)


# speedup vs baseline: 1.5342x; 1.5342x over previous
import jax
import jax.numpy as jnp
from jax.experimental import pallas as pl
from jax.experimental.pallas import tpu as pltpu

_GS = 16
_K = _GS + 1
_S = 128


def _knn_body(xq_ref, xr_ref, dist_ref, idx_ref):
    b = pl.program_id(0)
    xq = xq_ref[0]
    xr = xr_ref[0]
    Q = xq.shape[0]
    N = xr.shape[1]
    C = N // _S

    q0 = xq[:, 0:1]
    q1 = xq[:, 1:2]
    q2 = xq[:, 2:3]
    sq_q = q0 * q0 + q1 * q1 + q2 * q2
    inner = jnp.dot(xq.astype(jnp.bfloat16), xr.astype(jnp.bfloat16),
                    preferred_element_type=jnp.float32)

    inf = jnp.float32(jnp.inf)
    big = jnp.float32(N)
    lane = jax.lax.broadcasted_iota(
        jnp.int32, (1, _S), 1).astype(jnp.float32)
    cidx = jax.lax.broadcasted_iota(
        jnp.int32, (1, C), 1).astype(jnp.float32)

    chunks = []
    cms = []
    acms = []
    for c in range(C):
        r = xr[:, c * _S:(c + 1) * _S]
        r0 = r[0:1, :]
        r1 = r[1:2, :]
        r2 = r[2:3, :]
        sq_r = r0 * r0 + r1 * r1 + r2 * r2
        innr = inner[:, c * _S:(c + 1) * _S]
        d2c = jnp.maximum(sq_q + sq_r - 2.0 * innr, 0.0)
        gcol = lane + jnp.float32(c * _S)
        m = jnp.min(d2c, axis=1, keepdims=True)
        a = jnp.min(jnp.where(d2c == m, gcol, big), axis=1, keepdims=True)
        chunks.append(d2c)
        cms.append(m)
        acms.append(a)
    cm = jnp.concatenate(cms, axis=1)
    acm = jnp.concatenate(acms, axis=1)

    dists = []
    idxs = []
    for j in range(_K):
        m = jnp.min(cm, axis=1, keepdims=True)
        amin = jnp.min(jnp.where(cm == m, acm, big),
                       axis=1, keepdims=True)
        if j > 0:
            dists.append(jnp.sqrt(m))
            idxs.append(amin)
        if j == _K - 1:
            break
        cstar = jnp.floor(amin * jnp.float32(1.0 / _S))
        acc = chunks[0]
        for c in range(1, C):
            acc = jnp.where(cstar == jnp.float32(c), chunks[c], acc)
        gcol = cstar * jnp.float32(_S) + lane
        keep = (acc > m) | ((acc == m) & (gcol > amin))
        eff = jnp.where(keep, acc, inf)
        nm = jnp.min(eff, axis=1, keepdims=True)
        na = jnp.min(jnp.where(eff == nm, gcol, big), axis=1, keepdims=True)
        sel = cidx == cstar
        cm = jnp.where(sel, nm, cm)
        acm = jnp.where(sel, na, acm)
    dist_ref[0] = jnp.concatenate(dists, axis=1)
    idx = jnp.concatenate(idxs, axis=1) + jnp.float32(N) * b.astype(jnp.float32)
    idx_ref[0] = idx.astype(jnp.int32)


def kernel(xyz):
    B, N, _ = xyz.shape
    Q = 512
    xt = xyz.transpose(0, 2, 1)
    dist, idx = pl.pallas_call(
        _knn_body,
        grid=(B, N // Q),
        in_specs=[
            pl.BlockSpec((1, Q, 3), lambda b, q: (b, q, 0)),
            pl.BlockSpec((1, 3, N), lambda b, q: (b, 0, 0)),
        ],
        out_specs=[
            pl.BlockSpec((1, Q, _GS), lambda b, q: (b, q, 0)),
            pl.BlockSpec((1, Q, _GS), lambda b, q: (b, q, 0)),
        ],
        out_shape=[
            jax.ShapeDtypeStruct((B, N, _GS), jnp.float32),
            jax.ShapeDtypeStruct((B, N, _GS), jnp.int32),
        ],
        compiler_params=pltpu.CompilerParams(
            dimension_semantics=("parallel", "parallel")),
    )(xyz, xt)
    return dist, idx.reshape(-1)
